# baseline (speedup 1.0000x reference)
import jax
import jax.numpy as jnp
from jax import lax
from jax.experimental import pallas as pl
from jax.experimental.pallas import tpu as pltpu
from jax.experimental.pallas import tpu_sc as plsc

NX = 56
IN_SUB = 896
OUT_SUB = 4 * IN_SUB
HALF_OUT = OUT_SUB // 2
PAIR_IN = 2 * NX * NX
PAIR_OUT = 4 * PAIR_IN
NSUB = 4 * 24 * 4 * 7
NWORKERS = 32
SUB_PER_W = NSUB // NWORKERS
NBUF = 3


def _decode(k):
    b = k // 672
    r = k % 672
    ci = r // 28
    r2 = r % 28
    t2 = r2 // 7
    s = r2 % 7
    return b, 8 * ci, PAIR_IN * t2 + IN_SUB * s, PAIR_OUT * t2 + OUT_SUB * s


def _unpool_body(x_hbm, out_hbm, in0, in1, in2, ou0, ou1, ou2,
                 si0, si1, si2, so0, so1, so2):
    ins = [in0, in1, in2]
    outs = [ou0, ou1, ou2]
    sis = [si0, si1, si2]
    sos = [so0, so1, so2]
    c = lax.axis_index("c")
    s = lax.axis_index("s")
    wid = s * 2 + c
    base = wid * SUB_PER_W
    pat = lax.shift_right_logical(lax.iota(jnp.int32, 16), 1)

    def start_in(k, slot):
        b, c0, v0, _ = _decode(base + k)
        pltpu.async_copy(
            x_hbm.at[b, pl.ds(c0, 8), pl.ds(v0, IN_SUB)], ins[slot], sis[slot])

    def wait_in(k, slot):
        b, c0, v0, _ = _decode(base + k)
        pltpu.make_async_copy(
            x_hbm.at[b, pl.ds(c0, 8), pl.ds(v0, IN_SUB)],
            ins[slot], sis[slot]).wait()

    def start_out_half(k, slot, h):
        b, c0, _, o0 = _decode(base + k)
        pltpu.async_copy(
            outs[slot].at[:, pl.ds(h * HALF_OUT, HALF_OUT)],
            out_hbm.at[b, pl.ds(c0, 8), pl.ds(o0 + h * HALF_OUT, HALF_OUT)],
            sos[slot])

    def wait_out(k, slot):
        b, c0, _, o0 = _decode(base + k)
        for h in range(2):
            pltpu.make_async_copy(
                outs[slot].at[:, pl.ds(h * HALF_OUT, HALF_OUT)],
                out_hbm.at[b, pl.ds(c0, 8),
                           pl.ds(o0 + h * HALF_OUT, HALF_OUT)],
                sos[slot]).wait()

    for slot in range(NBUF):
        start_in(slot, slot)

    def step(g, carry):
        for slot in range(NBUF):
            k = g + slot
            wait_in(k, slot)

            @pl.when(g > 0)
            def _wait_out():
                wait_out(k - NBUF, slot)

            for h in range(2):
                for cc in range(8):
                    cv = jnp.full((16,), cc, jnp.int32)

                    @plsc.parallel_loop(8 * h, 8 * h + 8, unroll=4)
                    def q_body(q):
                        rb_out = 224 * q
                        for o in range(7):
                            idx = 56 * q + 8 * o + pat
                            v = plsc.load_gather(ins[slot], [cv, idx])
                            outs[slot][cc, pl.ds(rb_out + 16 * o, 16)] = v
                            outs[slot][cc,
                                       pl.ds(rb_out + 112 + 16 * o, 16)] = v

                start_out_half(k, slot, h)

            @pl.when(g + NBUF < SUB_PER_W)
            def _start_next_in():
                start_in(k + NBUF, slot)
        return carry

    lax.fori_loop(0, SUB_PER_W // NBUF, lambda i, c2: step(i * NBUF, c2), 0)

    for slot in range(NBUF):
        wait_out(SUB_PER_W - NBUF + slot, slot)


@jax.jit
def kernel(x):
    mesh = plsc.VectorSubcoreMesh(core_axis_name="c", subcore_axis_name="s")
    run = pl.kernel(
        _unpool_body,
        out_type=jax.ShapeDtypeStruct((4, 192, 100352), jnp.float32),
        mesh=mesh,
        scratch_types=(
            [pltpu.VMEM((8, IN_SUB), jnp.float32)] * NBUF
            + [pltpu.VMEM((8, OUT_SUB), jnp.float32)] * NBUF
            + [pltpu.SemaphoreType.DMA] * (2 * NBUF)
        ),
        compiler_params=pltpu.CompilerParams(needs_layout_passes=False),
    )
    return run(x)


# speedup vs baseline: 1.2534x; 1.2534x over previous
import jax
import jax.numpy as jnp
from jax import lax
from jax.experimental import pallas as pl
from jax.experimental.pallas import tpu as pltpu
from jax.experimental.pallas import tpu_sc as plsc

NX = 56
IN_SUB = 896
OUT_SUB = 4 * IN_SUB
PAIR_IN = 2 * NX * NX
PAIR_OUT = 4 * PAIR_IN
NSUB = 4 * 24 * 4 * 7
NWORKERS = 32
SUB_PER_W = NSUB // NWORKERS


def _decode(k):
    b = k // 672
    r = k % 672
    ci = r // 28
    r2 = r % 28
    t2 = r2 // 7
    s = r2 % 7
    return b, 8 * ci, PAIR_IN * t2 + IN_SUB * s, PAIR_OUT * t2 + OUT_SUB * s


def _unpool_body(x_hbm, out_hbm, in0, in1, in2, ou0, ou1, ou2,
                 si0, si1, si2, so0, so1, so2):
    ins = [in0, in1, in2]
    outs = [ou0, ou1, ou2]
    sis = [si0, si1, si2]
    sos = [so0, so1, so2]
    c = lax.axis_index("c")
    s = lax.axis_index("s")
    wid = s * 2 + c
    base = wid * SUB_PER_W
    pat = lax.shift_right_logical(lax.iota(jnp.int32, 16), 1)

    def start_in(k, slot):
        b, c0, v0, _ = _decode(base + k)
        pltpu.async_copy(
            x_hbm.at[b, pl.ds(c0, 8), pl.ds(v0, IN_SUB)], ins[slot], sis[slot])

    def wait_in(k, slot):
        b, c0, v0, _ = _decode(base + k)
        pltpu.make_async_copy(
            x_hbm.at[b, pl.ds(c0, 8), pl.ds(v0, IN_SUB)],
            ins[slot], sis[slot]).wait()

    def start_out(k, slot):
        b, c0, _, o0 = _decode(base + k)
        pltpu.async_copy(
            outs[slot], out_hbm.at[b, pl.ds(c0, 8), pl.ds(o0, OUT_SUB)],
            sos[slot])

    def wait_out(k, slot):
        b, c0, _, o0 = _decode(base + k)
        pltpu.make_async_copy(
            outs[slot], out_hbm.at[b, pl.ds(c0, 8), pl.ds(o0, OUT_SUB)],
            sos[slot]).wait()

    for slot in range(3):
        start_in(slot, slot)

    def step(g, carry):
        for slot in range(3):
            k = g + slot
            wait_in(k, slot)

            @pl.when(g > 0)
            def _wait_out():
                wait_out(k - 3, slot)

            for cc in range(8):
                cv = jnp.full((16,), cc, jnp.int32)

                @plsc.parallel_loop(0, 16, unroll=4)
                def q_body(q):
                    rb_out = 224 * q
                    for o in range(7):
                        idx = 56 * q + 8 * o + pat
                        v = plsc.load_gather(ins[slot], [cv, idx])
                        outs[slot][cc, pl.ds(rb_out + 16 * o, 16)] = v
                        outs[slot][cc, pl.ds(rb_out + 112 + 16 * o, 16)] = v

            start_out(k, slot)

            @pl.when(g + 3 < SUB_PER_W)
            def _start_next_in():
                start_in(k + 3, slot)
        return carry

    lax.fori_loop(0, SUB_PER_W // 3, lambda i, c2: step(i * 3, c2), 0)

    for slot in range(3):
        wait_out(SUB_PER_W - 3 + slot, slot)


@jax.jit
def kernel(x):
    mesh = plsc.VectorSubcoreMesh(core_axis_name="c", subcore_axis_name="s")
    run = pl.kernel(
        _unpool_body,
        out_type=jax.ShapeDtypeStruct((4, 192, 100352), jnp.float32),
        mesh=mesh,
        scratch_types=(
            [pltpu.VMEM((8, IN_SUB), jnp.float32)] * 3
            + [pltpu.VMEM((8, OUT_SUB), jnp.float32)] * 3
            + [pltpu.SemaphoreType.DMA] * 6
        ),
        compiler_params=pltpu.CompilerParams(needs_layout_passes=False),
    )
    return run(x)


# speedup vs baseline: 1.4250x; 1.1369x over previous
import jax
import jax.numpy as jnp
from jax import lax
from jax.experimental import pallas as pl
from jax.experimental.pallas import tpu as pltpu
from jax.experimental.pallas import tpu_sc as plsc

NX = 56
IN_SUB = 896
OUT_SUB = 4 * IN_SUB
PAIR_IN = 2 * NX * NX
PAIR_OUT = 4 * PAIR_IN
NSUB = 4 * 24 * 4 * 7
NWORKERS = 32
SUB_PER_W = NSUB // NWORKERS


def _decode(k):
    b = k // 672
    r = k % 672
    ci = r // 28
    r2 = r % 28
    t2 = r2 // 7
    s = r2 % 7
    return b, 8 * ci, PAIR_IN * t2 + IN_SUB * s, PAIR_OUT * t2 + OUT_SUB * s


def _unpool_body(x_hbm, out_hbm, in0, in1, ou0, ou1, si0, si1, so0, so1):
    ins = [in0, in1]
    outs = [ou0, ou1]
    sis = [si0, si1]
    sos = [so0, so1]
    c = lax.axis_index("c")
    s = lax.axis_index("s")
    wid = s * 2 + c
    base = wid * SUB_PER_W
    pat = lax.shift_right_logical(lax.iota(jnp.int32, 16), 1)

    def start_in(k, slot):
        b, c0, v0, _ = _decode(base + k)
        pltpu.async_copy(
            x_hbm.at[b, pl.ds(c0, 8), pl.ds(v0, IN_SUB)], ins[slot], sis[slot])

    def wait_in(k, slot):
        b, c0, v0, _ = _decode(base + k)
        pltpu.make_async_copy(
            x_hbm.at[b, pl.ds(c0, 8), pl.ds(v0, IN_SUB)],
            ins[slot], sis[slot]).wait()

    def start_out(k, slot):
        b, c0, _, o0 = _decode(base + k)
        pltpu.async_copy(
            outs[slot], out_hbm.at[b, pl.ds(c0, 8), pl.ds(o0, OUT_SUB)],
            sos[slot])

    def wait_out(k, slot):
        b, c0, _, o0 = _decode(base + k)
        pltpu.make_async_copy(
            outs[slot], out_hbm.at[b, pl.ds(c0, 8), pl.ds(o0, OUT_SUB)],
            sos[slot]).wait()

    for slot in range(2):
        start_in(slot, slot)

    def step(g, carry):
        for slot in range(2):
            k = g + slot
            wait_in(k, slot)

            @pl.when(g > 0)
            def _wait_out():
                wait_out(k - 2, slot)

            for cc in range(8):
                cv = jnp.full((16,), cc, jnp.int32)

                @plsc.parallel_loop(0, 16, unroll=8)
                def q_body(q):
                    rb_out = 224 * q
                    for o in range(7):
                        idx = 56 * q + 8 * o + pat
                        v = plsc.load_gather(ins[slot], [cv, idx])
                        outs[slot][cc, pl.ds(rb_out + 16 * o, 16)] = v
                        outs[slot][cc, pl.ds(rb_out + 112 + 16 * o, 16)] = v

            start_out(k, slot)

            @pl.when(g + 2 < SUB_PER_W)
            def _start_next_in():
                start_in(k + 2, slot)
        return carry

    lax.fori_loop(0, SUB_PER_W // 2, lambda i, c2: step(i * 2, c2), 0)

    for slot in range(2):
        wait_out(SUB_PER_W - 2 + slot, slot)


@jax.jit
def kernel(x):
    mesh = plsc.VectorSubcoreMesh(core_axis_name="c", subcore_axis_name="s")
    run = pl.kernel(
        _unpool_body,
        out_type=jax.ShapeDtypeStruct((4, 192, 100352), jnp.float32),
        mesh=mesh,
        scratch_types=[
            pltpu.VMEM((8, IN_SUB), jnp.float32),
            pltpu.VMEM((8, IN_SUB), jnp.float32),
            pltpu.VMEM((8, OUT_SUB), jnp.float32),
            pltpu.VMEM((8, OUT_SUB), jnp.float32),
            pltpu.SemaphoreType.DMA,
            pltpu.SemaphoreType.DMA,
            pltpu.SemaphoreType.DMA,
            pltpu.SemaphoreType.DMA,
        ],
        compiler_params=pltpu.CompilerParams(needs_layout_passes=False),
    )
    return run(x)
